# Initial kernel scaffold; baseline (speedup 1.0000x reference)
import jax
import jax.numpy as jnp
from jax import lax
from jax.experimental import pallas as pl
from jax.experimental.pallas import tpu as pltpu
from jax.experimental.pallas import tpu_sc as plsc

N = 10000
D = 128
E = 320000

NC = 2
NS = 16
NW = NC * NS
NPAD = 10240
EPAD = 327680
EPT = EPAD // NW
K = 128
CHUNKS = EPT // K
RPT = NPAD // NS

_MESH = plsc.VectorSubcoreMesh(
    core_axis_name="c", subcore_axis_name="s", num_cores=NC, num_subcores=NS
)

_ZV = jnp.zeros((16,), jnp.float32)
_OV = jnp.ones((16,), jnp.float32)


def _deg_body(dst_hbm, degp_hbm, idx_v, ones_v, zrow_v, hist_sh):
    c = lax.axis_index("c")
    s = lax.axis_index("s")

    def fill_o(i, carry):
        ones_v[pl.ds(i * 16, 16)] = _OV
        return carry

    lax.fori_loop(0, K // 16, fill_o, 0)

    def fill_z(i, carry):
        zrow_v[pl.ds(i * 16, 16)] = _ZV
        return carry

    lax.fori_loop(0, RPT // 16, fill_z, 0)

    my_rows = pl.multiple_of(s * RPT, 8)
    pltpu.sync_copy(zrow_v, hist_sh.at[pl.ds(my_rows, RPT)])
    plsc.subcore_barrier()

    base = pl.multiple_of((c * NS + s) * EPT, 8)

    def chunk(k, carry):
        off = pl.multiple_of(base + k * K, 8)
        pltpu.sync_copy(dst_hbm.at[pl.ds(off, K)], idx_v)
        pltpu.sync_copy(ones_v, hist_sh.at[idx_v], add=True)
        return carry

    lax.fori_loop(0, CHUNKS, chunk, 0)
    plsc.subcore_barrier()
    pltpu.sync_copy(hist_sh.at[pl.ds(my_rows, RPT)],
                    degp_hbm.at[c, pl.ds(my_rows, RPT)])


_deg_kernel = pl.kernel(
    _deg_body,
    out_type=jax.ShapeDtypeStruct((NC, NPAD), jnp.float32),
    mesh=_MESH,
    scratch_types=[
        pltpu.VMEM((K,), jnp.int32),
        pltpu.VMEM((K,), jnp.float32),
        pltpu.VMEM((RPT,), jnp.float32),
        pltpu.VMEM_SHARED((NPAD,), jnp.float32),
    ],
)


def _agg_body(src_hbm, dst_hbm, y_hbm, aggp_hbm, sidx_v, didx_v, rows_v,
              agg_sh, sem):
    c = lax.axis_index("c")
    s = lax.axis_index("s")

    def fz(i, carry):
        def fz2(j, carry2):
            rows_v[i, pl.ds(j * 16, 16)] = _ZV
            return carry2
        return lax.fori_loop(0, D // 16, fz2, carry)

    lax.fori_loop(0, K, fz, 0)

    my_rows = pl.multiple_of(s * RPT, 8)

    def fzd(j, carry):
        off = pl.multiple_of(my_rows + j * K, 8)
        pltpu.sync_copy(rows_v, agg_sh.at[pl.ds(off, K)])
        return carry

    lax.fori_loop(0, RPT // K, fzd, 0)
    plsc.subcore_barrier()

    base = pl.multiple_of((c * NS + s) * EPT, 8)

    def chunk(k, carry):
        off = pl.multiple_of(base + k * K, 8)
        pltpu.sync_copy(src_hbm.at[pl.ds(off, K)], sidx_v)
        pltpu.sync_copy(dst_hbm.at[pl.ds(off, K)], didx_v)
        pltpu.async_copy(y_hbm.at[sidx_v], rows_v, sem).wait()
        pltpu.sync_copy(rows_v, agg_sh.at[didx_v], add=True)
        return carry

    lax.fori_loop(0, CHUNKS, chunk, 0)
    plsc.subcore_barrier()
    pltpu.sync_copy(agg_sh.at[pl.ds(my_rows, RPT)],
                    aggp_hbm.at[c, pl.ds(my_rows, RPT)])


_agg_kernel = pl.kernel(
    _agg_body,
    out_type=jax.ShapeDtypeStruct((NC, NPAD, D), jnp.float32),
    mesh=_MESH,
    scratch_types=[
        pltpu.VMEM((K,), jnp.int32),
        pltpu.VMEM((K,), jnp.int32),
        pltpu.VMEM((K, D), jnp.float32),
        pltpu.VMEM_SHARED((NPAD, D), jnp.float32),
        pltpu.SemaphoreType.DMA,
    ],
)


_BB = 512

def _y_body(x_ref, w_ref, degp_ref, y_ref):
    xw = jnp.dot(x_ref[...], w_ref[...], preferred_element_type=jnp.float32)
    deg = degp_ref[0, :] + degp_ref[1, :] + 1.0
    dinv = lax.rsqrt(deg)
    y_ref[...] = xw * dinv[:, None]


_y_call = pl.pallas_call(
    _y_body,
    grid=(NPAD // _BB,),
    in_specs=[
        pl.BlockSpec((_BB, D), lambda i: (i, 0)),
        pl.BlockSpec((D, D), lambda i: (0, 0)),
        pl.BlockSpec((NC, _BB), lambda i: (0, i)),
    ],
    out_specs=pl.BlockSpec((_BB, D), lambda i: (i, 0)),
    out_shape=jax.ShapeDtypeStruct((NPAD, D), jnp.float32),
)


_OB = 400

def _out_body(x_ref, aggp_ref, y_ref, degp_ref, b_ref, o_ref):
    ssum = aggp_ref[0] + aggp_ref[1] + y_ref[...]
    deg = degp_ref[0, :] + degp_ref[1, :] + 1.0
    dinv = lax.rsqrt(deg)
    conv = ssum * dinv[:, None] + b_ref[...]
    o_ref[...] = x_ref[...] + jnp.maximum(conv, 0.0)


_out_call = pl.pallas_call(
    _out_body,
    grid=(N // _OB,),
    in_specs=[
        pl.BlockSpec((_OB, D), lambda i: (i, 0)),
        pl.BlockSpec((NC, _OB, D), lambda i: (0, i, 0)),
        pl.BlockSpec((_OB, D), lambda i: (i, 0)),
        pl.BlockSpec((NC, _OB), lambda i: (0, i)),
        pl.BlockSpec((1, D), lambda i: (0, 0)),
    ],
    out_specs=pl.BlockSpec((_OB, D), lambda i: (i, 0)),
    out_shape=jax.ShapeDtypeStruct((N, D), jnp.float32),
)


def kernel(x, edge_index, W, b):
    src = edge_index[0]
    dst = edge_index[1]
    pad = (N + (jnp.arange(EPAD - E, dtype=jnp.int32) % (NPAD - N))).astype(
        jnp.int32)
    src_p = jnp.concatenate([src, pad])
    dst_p = jnp.concatenate([dst, pad])
    x_p = jnp.concatenate([x, jnp.zeros((NPAD - N, D), x.dtype)], axis=0)

    degp = _deg_kernel(dst_p)
    y = _y_call(x_p, W, degp)
    aggp = _agg_kernel(src_p, dst_p, y)
    return _out_call(x, aggp, y, degp, b.reshape(1, D))


# speedup vs baseline: 19.5283x; 19.5283x over previous
import jax
import jax.numpy as jnp
from jax import lax
from jax.experimental import pallas as pl
from jax.experimental.pallas import tpu as pltpu
from jax.experimental.pallas import tpu_sc as plsc

N = 10000
D = 128
E = 320000

NC = 2
NS = 16
NW = NC * NS
NPAD = 10240
EPAD = 327680
EPT = EPAD // NW
K = 128
CHUNKS = EPT // K
RPT = NPAD // NS

_MESH = plsc.VectorSubcoreMesh(
    core_axis_name="c", subcore_axis_name="s", num_cores=NC, num_subcores=NS
)


def _deg_body(dst_hbm, degp_hbm, idx_v, ones_v, zrow_v, hist_sh):
    c = lax.axis_index("c")
    s = lax.axis_index("s")
    _ZV = jnp.zeros((16,), jnp.float32)
    _OV = jnp.ones((16,), jnp.float32)

    def fill_o(i, carry):
        ones_v[pl.ds(i * 16, 16)] = _OV
        return carry

    lax.fori_loop(0, K // 16, fill_o, 0)

    def fill_z(i, carry):
        zrow_v[pl.ds(i * 16, 16)] = _ZV
        return carry

    lax.fori_loop(0, RPT // 16, fill_z, 0)

    my_rows = pl.multiple_of(s * RPT, 8)
    pltpu.sync_copy(zrow_v, hist_sh.at[pl.ds(my_rows, RPT)])
    plsc.subcore_barrier()

    base = pl.multiple_of((c * NS + s) * EPT, 8)

    def chunk(k, carry):
        off = pl.multiple_of(base + k * K, 8)
        pltpu.sync_copy(dst_hbm.at[pl.ds(off, K)], idx_v)
        pltpu.sync_copy(ones_v, hist_sh.at[idx_v], add=True)
        return carry

    lax.fori_loop(0, CHUNKS, chunk, 0)
    plsc.subcore_barrier()
    pltpu.sync_copy(hist_sh.at[pl.ds(my_rows, RPT)],
                    degp_hbm.at[c, pl.ds(my_rows, RPT)])


_deg_kernel = pl.kernel(
    _deg_body,
    out_type=jax.ShapeDtypeStruct((NC, NPAD), jnp.float32),
    mesh=_MESH,
    scratch_types=[
        pltpu.VMEM((K,), jnp.int32),
        pltpu.VMEM((K,), jnp.float32),
        pltpu.VMEM((RPT,), jnp.float32),
        pltpu.VMEM_SHARED((NPAD,), jnp.float32),
    ],
)


def _agg_body(src_hbm, dst_hbm, y_hbm, aggp_hbm, sidx_v, didx_v, rows_v,
              agg_sh, sem):
    c = lax.axis_index("c")
    s = lax.axis_index("s")
    _ZV = jnp.zeros((16,), jnp.float32)

    def fz(i, carry):
        def fz2(j, carry2):
            rows_v[i, pl.ds(j * 16, 16)] = _ZV
            return carry2
        return lax.fori_loop(0, D // 16, fz2, carry)

    lax.fori_loop(0, K, fz, 0)

    my_rows = pl.multiple_of(s * RPT, 8)

    def fzd(j, carry):
        off = pl.multiple_of(my_rows + j * K, 8)
        pltpu.sync_copy(rows_v, agg_sh.at[pl.ds(off, K)])
        return carry

    lax.fori_loop(0, RPT // K, fzd, 0)
    plsc.subcore_barrier()

    base = pl.multiple_of((c * NS + s) * EPT, 8)

    def chunk(k, carry):
        off = pl.multiple_of(base + k * K, 8)
        pltpu.sync_copy(src_hbm.at[pl.ds(off, K)], sidx_v)
        pltpu.sync_copy(dst_hbm.at[pl.ds(off, K)], didx_v)
        pltpu.async_copy(y_hbm.at[sidx_v], rows_v, sem).wait()
        pltpu.sync_copy(rows_v, agg_sh.at[didx_v], add=True)
        return carry

    lax.fori_loop(0, CHUNKS, chunk, 0)
    plsc.subcore_barrier()
    pltpu.sync_copy(agg_sh.at[pl.ds(my_rows, RPT)],
                    aggp_hbm.at[c, pl.ds(my_rows, RPT)])


_agg_kernel = pl.kernel(
    _agg_body,
    out_type=jax.ShapeDtypeStruct((NC, NPAD, D), jnp.float32),
    mesh=_MESH,
    scratch_types=[
        pltpu.VMEM((K,), jnp.int32),
        pltpu.VMEM((K,), jnp.int32),
        pltpu.VMEM((K, D), jnp.float32),
        pltpu.VMEM_SHARED((NPAD, D), jnp.float32),
        pltpu.SemaphoreType.DMA,
    ],
)


_BB = 512

def _y_body(x_ref, w_ref, degp_ref, y_ref):
    xw = jnp.dot(x_ref[...], w_ref[...], preferred_element_type=jnp.float32)
    deg = degp_ref[0, :] + degp_ref[1, :] + 1.0
    dinv = lax.rsqrt(deg)
    y_ref[...] = xw * dinv[:, None]


_y_call = pl.pallas_call(
    _y_body,
    grid=(NPAD // _BB,),
    in_specs=[
        pl.BlockSpec((_BB, D), lambda i: (i, 0)),
        pl.BlockSpec((D, D), lambda i: (0, 0)),
        pl.BlockSpec((NC, _BB), lambda i: (0, i)),
    ],
    out_specs=pl.BlockSpec((_BB, D), lambda i: (i, 0)),
    out_shape=jax.ShapeDtypeStruct((NPAD, D), jnp.float32),
)


_OB = 512

def _out_body(x_ref, aggp_ref, y_ref, degp_ref, b_ref, o_ref):
    ssum = aggp_ref[0] + aggp_ref[1] + y_ref[...]
    deg = degp_ref[0, :] + degp_ref[1, :] + 1.0
    dinv = lax.rsqrt(deg)
    conv = ssum * dinv[:, None] + b_ref[...]
    o_ref[...] = x_ref[...] + jnp.maximum(conv, 0.0)


_out_call = pl.pallas_call(
    _out_body,
    grid=(pl.cdiv(N, _OB),),
    in_specs=[
        pl.BlockSpec((_OB, D), lambda i: (i, 0)),
        pl.BlockSpec((NC, _OB, D), lambda i: (0, i, 0)),
        pl.BlockSpec((_OB, D), lambda i: (i, 0)),
        pl.BlockSpec((NC, _OB), lambda i: (0, i)),
        pl.BlockSpec((1, D), lambda i: (0, 0)),
    ],
    out_specs=pl.BlockSpec((_OB, D), lambda i: (i, 0)),
    out_shape=jax.ShapeDtypeStruct((N, D), jnp.float32),
)


def kernel(x, edge_index, W, b):
    src = edge_index[0]
    dst = edge_index[1]
    pad = (N + (jnp.arange(EPAD - E, dtype=jnp.int32) % (NPAD - N))).astype(
        jnp.int32)
    src_p = jnp.concatenate([src, pad])
    dst_p = jnp.concatenate([dst, pad])
    x_p = jnp.concatenate([x, jnp.zeros((NPAD - N, D), x.dtype)], axis=0)

    degp = _deg_kernel(dst_p)
    y = _y_call(x_p, W, degp)
    aggp = _agg_kernel(src_p, dst_p, y)
    return _out_call(x, aggp, y, degp, b.reshape(1, D))


# speedup vs baseline: 38.9104x; 1.9925x over previous
import jax
import jax.numpy as jnp
from jax import lax
from jax.experimental import pallas as pl
from jax.experimental.pallas import tpu as pltpu
from jax.experimental.pallas import tpu_sc as plsc

N = 10000
D = 128
E = 320000

NC = 2
NS = 16
NW = NC * NS
NPAD = 10240
EPAD = 327680
EPT = EPAD // NW
K = 128
CHUNKS = EPT // K
RPT = NPAD // NS

_MESH = plsc.VectorSubcoreMesh(
    core_axis_name="c", subcore_axis_name="s", num_cores=NC, num_subcores=NS
)


def _deg_body(dst2_hbm, degp_hbm, didx_v, ones_v, zrow_v, hist_sh, asem):
    c = lax.axis_index("c")
    s = lax.axis_index("s")
    _ZV = jnp.zeros((16,), jnp.float32)
    _OV = jnp.ones((16,), jnp.float32)

    def fill_o(i, carry):
        ones_v[pl.ds(i * 16, 16)] = _OV
        return carry

    lax.fori_loop(0, K // 16, fill_o, 0)

    def fill_z(i, carry):
        zrow_v[pl.ds(i * 16, 16)] = _ZV
        return carry

    lax.fori_loop(0, RPT // 16, fill_z, 0)

    my_rows = pl.multiple_of(s * RPT, 8)
    pltpu.sync_copy(zrow_v, hist_sh.at[pl.ds(my_rows, RPT)])
    row0 = pl.multiple_of((c * NS + s) * CHUNKS, 8)
    pltpu.sync_copy(dst2_hbm.at[pl.ds(row0, CHUNKS)], didx_v)
    plsc.subcore_barrier()

    def fire(k, carry):
        pltpu.async_copy(ones_v, hist_sh.at[didx_v.at[k]], asem, add=True)
        return carry

    lax.fori_loop(0, CHUNKS, fire, 0)

    def drain(k, carry):
        pltpu.make_async_copy(ones_v, hist_sh.at[didx_v.at[0]], asem).wait()
        return carry

    lax.fori_loop(0, CHUNKS, drain, 0)
    plsc.subcore_barrier()
    pltpu.sync_copy(hist_sh.at[pl.ds(my_rows, RPT)],
                    degp_hbm.at[c, pl.ds(my_rows, RPT)])


_deg_kernel = pl.kernel(
    _deg_body,
    out_type=jax.ShapeDtypeStruct((NC, NPAD), jnp.float32),
    mesh=_MESH,
    scratch_types=[
        pltpu.VMEM((CHUNKS, K), jnp.int32),
        pltpu.VMEM((K,), jnp.float32),
        pltpu.VMEM((RPT,), jnp.float32),
        pltpu.VMEM_SHARED((NPAD,), jnp.float32),
        pltpu.SemaphoreType.DMA,
    ],
)


NBUF = 2
G = CHUNKS // NBUF

def _agg_body(packed_hbm, y_hbm, aggp_hbm, packed_v, sidx_v, didx_v, rows_v,
              agg_sh, *sems):
    gsem = sems[:NBUF]
    ssem = sems[NBUF:]
    c = lax.axis_index("c")
    s = lax.axis_index("s")
    _ZV = jnp.zeros((16,), jnp.float32)

    def fz(i, carry):
        def fz2(j, carry2):
            rows_v[0, i, pl.ds(j * 16, 16)] = _ZV
            return carry2
        return lax.fori_loop(0, D // 16, fz2, carry)

    lax.fori_loop(0, K, fz, 0)

    my_rows = pl.multiple_of(s * RPT, 8)

    def fzd(j, carry):
        off = pl.multiple_of(my_rows + j * K, 8)
        pltpu.sync_copy(rows_v.at[0], agg_sh.at[pl.ds(off, K)])
        return carry

    lax.fori_loop(0, RPT // K, fzd, 0)

    row0 = pl.multiple_of((c * NS + s) * CHUNKS, 8)
    pltpu.sync_copy(packed_hbm.at[pl.ds(row0, CHUNKS)], packed_v)
    plsc.subcore_barrier()

    def unpack(k, b):
        def up(j, carry):
            p = packed_v[k, pl.ds(j * 16, 16)]
            sidx_v[b, pl.ds(j * 16, 16)] = p & 0xFFFF
            didx_v[b, pl.ds(j * 16, 16)] = lax.shift_right_logical(p, 16)
            return carry
        lax.fori_loop(0, K // 16, up, 0)

    for b in range(NBUF):
        unpack(b, b)
        pltpu.async_copy(y_hbm.at[sidx_v.at[b]], rows_v.at[b], gsem[b])

    def outer(g, carry):
        for b in range(NBUF):
            k = g * NBUF + b
            pltpu.make_async_copy(y_hbm.at[sidx_v.at[b]], rows_v.at[b],
                                  gsem[b]).wait()
            pltpu.async_copy(rows_v.at[b], agg_sh.at[didx_v.at[b]], ssem[b],
                             add=True)
            pltpu.make_async_copy(rows_v.at[b], agg_sh.at[didx_v.at[b]],
                                  ssem[b]).wait()
            unpack(k + NBUF, b)
            pltpu.async_copy(y_hbm.at[sidx_v.at[b]], rows_v.at[b], gsem[b])
        return carry

    lax.fori_loop(0, G - 1, outer, 0)
    for b in range(NBUF):
        pltpu.make_async_copy(y_hbm.at[sidx_v.at[b]], rows_v.at[b],
                              gsem[b]).wait()
        pltpu.async_copy(rows_v.at[b], agg_sh.at[didx_v.at[b]], ssem[b],
                         add=True)
    for b in range(NBUF):
        pltpu.make_async_copy(rows_v.at[b], agg_sh.at[didx_v.at[b]],
                              ssem[b]).wait()

    plsc.subcore_barrier()
    pltpu.sync_copy(agg_sh.at[pl.ds(my_rows, RPT)],
                    aggp_hbm.at[c, pl.ds(my_rows, RPT)])


_agg_kernel = pl.kernel(
    _agg_body,
    out_type=jax.ShapeDtypeStruct((NC, NPAD, D), jnp.float32),
    mesh=_MESH,
    scratch_types=[
        pltpu.VMEM((CHUNKS, K), jnp.int32),
        pltpu.VMEM((NBUF, K), jnp.int32),
        pltpu.VMEM((NBUF, K), jnp.int32),
        pltpu.VMEM((NBUF, K, D), jnp.float32),
        pltpu.VMEM_SHARED((NPAD, D), jnp.float32),
    ] + [pltpu.SemaphoreType.DMA] * (2 * NBUF),
)


_BB = 512

def _y_body(x_ref, w_ref, degp_ref, y_ref):
    xw = jnp.dot(x_ref[...], w_ref[...], preferred_element_type=jnp.float32)
    deg = degp_ref[0, :] + degp_ref[1, :] + 1.0
    dinv = lax.rsqrt(deg)
    y_ref[...] = xw * dinv[:, None]


_y_call = pl.pallas_call(
    _y_body,
    grid=(NPAD // _BB,),
    in_specs=[
        pl.BlockSpec((_BB, D), lambda i: (i, 0)),
        pl.BlockSpec((D, D), lambda i: (0, 0)),
        pl.BlockSpec((NC, _BB), lambda i: (0, i)),
    ],
    out_specs=pl.BlockSpec((_BB, D), lambda i: (i, 0)),
    out_shape=jax.ShapeDtypeStruct((NPAD, D), jnp.float32),
)


_OB = 512

def _out_body(x_ref, aggp_ref, y_ref, degp_ref, b_ref, o_ref):
    ssum = aggp_ref[0] + aggp_ref[1] + y_ref[...]
    deg = degp_ref[0, :] + degp_ref[1, :] + 1.0
    dinv = lax.rsqrt(deg)
    conv = ssum * dinv[:, None] + b_ref[...]
    o_ref[...] = x_ref[...] + jnp.maximum(conv, 0.0)


_out_call = pl.pallas_call(
    _out_body,
    grid=(pl.cdiv(N, _OB),),
    in_specs=[
        pl.BlockSpec((_OB, D), lambda i: (i, 0)),
        pl.BlockSpec((NC, _OB, D), lambda i: (0, i, 0)),
        pl.BlockSpec((_OB, D), lambda i: (i, 0)),
        pl.BlockSpec((NC, _OB), lambda i: (0, i)),
        pl.BlockSpec((1, D), lambda i: (0, 0)),
    ],
    out_specs=pl.BlockSpec((_OB, D), lambda i: (i, 0)),
    out_shape=jax.ShapeDtypeStruct((N, D), jnp.float32),
)


def kernel(x, edge_index, W, b):
    src = edge_index[0]
    dst = edge_index[1]
    pad = (N + (jnp.arange(EPAD - E, dtype=jnp.int32) % (NPAD - N))).astype(
        jnp.int32)
    src_p = jnp.concatenate([src, pad])
    dst_p = jnp.concatenate([dst, pad])
    packed = (src_p | (dst_p << 16)).reshape(EPAD // K, K)
    dst2 = dst_p.reshape(EPAD // K, K)
    x_p = jnp.concatenate([x, jnp.zeros((NPAD - N, D), x.dtype)], axis=0)

    degp = _deg_kernel(dst2)
    y = _y_call(x_p, W, degp)
    aggp = _agg_kernel(packed, y)
    return _out_call(x, aggp, y, degp, b.reshape(1, D))
